# baseline (speedup 1.0000x reference)
import functools

import jax
import jax.numpy as jnp
from jax import lax
from jax.experimental import pallas as pl
from jax.experimental.pallas import tpu as pltpu
from jax.experimental.pallas import tpu_sc as plsc

B = 4096
L = 50
D = 64
NC = 2
NS = 16
NW = NC * NS
BW = B // NW
LINES = 1000000 // 2


def kernel(user_id, items_ids, user_table, item_table):
    ut2 = user_table.reshape(LINES, 2 * D)
    it2 = item_table.reshape(LINES, 2 * D)
    iidx_t = items_ids.T

    mesh = plsc.VectorSubcoreMesh(
        core_axis_name="core", subcore_axis_name="subcore"
    )

    @functools.partial(
        pl.kernel,
        out_type=(
            jax.ShapeDtypeStruct((D, B), jnp.float32),
            jax.ShapeDtypeStruct((L, D, B), jnp.float32),
        ),
        mesh=mesh,
        scratch_types=[
            pltpu.VMEM((2, BW), jnp.int32),
            pltpu.VMEM((2, BW), jnp.int32),
            pltpu.VMEM((2, BW), jnp.int32),
            pltpu.VMEM((2 * BW, 2 * D), jnp.float32),
            pltpu.VMEM((2 * D, BW), jnp.float32),
            pltpu.SemaphoreType.DMA((2,)),
            pltpu.SemaphoreType.DMA((2,)),
            pltpu.SemaphoreType.DMA,
        ],
        compiler_params=pltpu.CompilerParams(
            use_tc_tiling_on_sc=True, needs_layout_passes=False
        ),
    )
    def run(ut_hbm, it_hbm, uid_hbm, iidx_hbm, uo_hbm, io_hbm,
            raw_v, lin_v, hof_v, lines_v, tb_v, gsem, ssem, isem):
        c = lax.axis_index("core")
        s = lax.axis_index("subcore")
        wid = s * NC + c
        b0 = wid * BW

        iota = lax.iota(jnp.int32, 16)

        def prep_idx(q):
            for i in range(BW // 16):
                v = raw_v.at[q][pl.ds(i * 16, 16)][...]
                lin_v.at[q][pl.ds(i * 16, 16)] = v >> 1
                hof_v.at[q][pl.ds(i * 16, 16)] = (v & 1) * D

        def fire_gather(table, q):
            pltpu.async_copy(
                table.at[lin_v.at[q]],
                lines_v.at[pl.ds(q * BW, BW)],
                gsem.at[q],
            )

        def wait_gather(table, q):
            pltpu.make_async_copy(
                table.at[lin_v.at[q]],
                lines_v.at[pl.ds(q * BW, BW)],
                gsem.at[q],
            ).wait()

        def extract(q):
            lines = lines_v.at[pl.ds(q * BW, BW)]
            tb = tb_v.at[pl.ds(q * D, D)]

            @pl.loop(0, BW // 16)
            def _(k0):
                rows = iota + k0 * 16
                cols0 = hof_v.at[q][pl.ds(k0 * 16, 16)][...]
                for f in range(D):
                    vec = plsc.load_gather(lines, [rows, cols0 + f])
                    tb.at[f][pl.ds(k0 * 16, 16)] = vec

        pltpu.sync_copy(uid_hbm.at[pl.ds(b0, BW)], raw_v.at[0])
        prep_idx(0)
        fire_gather(ut_hbm, 0)
        wait_gather(ut_hbm, 0)
        extract(0)
        pltpu.async_copy(
            tb_v.at[pl.ds(0, D)], uo_hbm.at[:, pl.ds(b0, BW)], ssem.at[0]
        )
        pltpu.make_async_copy(
            tb_v.at[pl.ds(0, D)], uo_hbm.at[:, pl.ds(b0, BW)], ssem.at[0]
        ).wait()

        def load_idx(l, q):
            pltpu.sync_copy(iidx_hbm.at[l, pl.ds(b0, BW)], raw_v.at[q])
            prep_idx(q)

        def fire_block_store(l, q):
            pltpu.async_copy(
                tb_v.at[pl.ds(q * D, D)],
                io_hbm.at[l, :, pl.ds(b0, BW)],
                ssem.at[q],
            )

        def wait_block_store(l, q):
            pltpu.make_async_copy(
                tb_v.at[pl.ds(q * D, D)],
                io_hbm.at[l, :, pl.ds(b0, BW)],
                ssem.at[q],
            ).wait()

        load_idx(0, 0)
        fire_gather(it_hbm, 0)

        @pl.loop(0, L, step=2)
        def _(l0):
            for q in (0, 1):
                l = l0 + q

                @pl.when(l + 1 < L)
                def _():
                    load_idx(l + 1, 1 - q)
                    fire_gather(it_hbm, 1 - q)

                wait_gather(it_hbm, q)

                @pl.when(l >= 2)
                def _():
                    wait_block_store(l - 2, q)

                extract(q)
                fire_block_store(l, q)

        wait_block_store(L - 2, 0)
        wait_block_store(L - 1, 1)

    user_t, item_t = run(ut2, it2, user_id, iidx_t)
    return user_t.T, jnp.transpose(item_t, (2, 0, 1))
